# Initial kernel scaffold; baseline (speedup 1.0000x reference)
import jax
import jax.numpy as jnp
from jax.experimental import pallas as pl


def kernel(x):
    raise NotImplementedError("write your pallas kernel here")


# speedup vs baseline: 3.2888x; 3.2888x over previous
import jax
import jax.numpy as jnp
from jax.experimental import pallas as pl
from jax.experimental.pallas import tpu as pltpu

ROWS_PER_BLOCK = 256
CHUNK = 128


def _cumsum_kernel(x_ref, o_ref):
    rows = x_ref.shape[0]
    ncols = x_ref.shape[1]
    nchunks = ncols // CHUNK
    row_i = jax.lax.broadcasted_iota(jnp.int32, (CHUNK, CHUNK), 0)
    col_i = jax.lax.broadcasted_iota(jnp.int32, (CHUNK, CHUNK), 1)
    tri = (row_i <= col_i).astype(jnp.float32)

    def body(c, carry):
        xc = x_ref[:, pl.ds(c * CHUNK, CHUNK)]
        local = jax.lax.dot(xc, tri, preferred_element_type=jnp.float32)
        o_ref[:, pl.ds(c * CHUNK, CHUNK)] = local + carry
        return carry + local[:, CHUNK - 1:CHUNK]

    carry0 = jnp.zeros((rows, 1), jnp.float32)
    jax.lax.fori_loop(0, nchunks, body, carry0)


def kernel(x):
    m, n = x.shape
    return pl.pallas_call(
        _cumsum_kernel,
        grid=(m // ROWS_PER_BLOCK,),
        in_specs=[pl.BlockSpec((ROWS_PER_BLOCK, n), lambda i: (i, 0))],
        out_specs=pl.BlockSpec((ROWS_PER_BLOCK, n), lambda i: (i, 0)),
        out_shape=jax.ShapeDtypeStruct((m, n), x.dtype),
        compiler_params=pltpu.CompilerParams(
            dimension_semantics=("parallel",)),
    )(x)


# speedup vs baseline: 6.5366x; 1.9875x over previous
import jax
import jax.numpy as jnp
from jax.experimental import pallas as pl
from jax.experimental.pallas import tpu as pltpu

ROWS_PER_BLOCK = 256
CHUNK = 128


def _cumsum_kernel(x_ref, o_ref):
    rows = x_ref.shape[0]
    ncols = x_ref.shape[1]
    nchunks = ncols // CHUNK
    row_i = jax.lax.broadcasted_iota(jnp.int32, (CHUNK, CHUNK), 0)
    col_i = jax.lax.broadcasted_iota(jnp.int32, (CHUNK, CHUNK), 1)
    tri = (row_i <= col_i).astype(jnp.float32)

    carry = jnp.zeros((rows, 1), jnp.float32)
    for c in range(nchunks):
        xc = x_ref[:, c * CHUNK:(c + 1) * CHUNK]
        local = jax.lax.dot(xc, tri, preferred_element_type=jnp.float32)
        o_ref[:, c * CHUNK:(c + 1) * CHUNK] = local + carry
        carry = carry + local[:, CHUNK - 1:CHUNK]


def kernel(x):
    m, n = x.shape
    return pl.pallas_call(
        _cumsum_kernel,
        grid=(m // ROWS_PER_BLOCK,),
        in_specs=[pl.BlockSpec((ROWS_PER_BLOCK, n), lambda i: (i, 0))],
        out_specs=pl.BlockSpec((ROWS_PER_BLOCK, n), lambda i: (i, 0)),
        out_shape=jax.ShapeDtypeStruct((m, n), x.dtype),
        compiler_params=pltpu.CompilerParams(
            dimension_semantics=("parallel",)),
    )(x)
